# baseline (speedup 1.0000x reference)
import functools

import jax
import jax.numpy as jnp
from jax import lax
from jax.experimental import pallas as pl
from jax.experimental.pallas import tpu as pltpu
from jax.experimental.pallas import tpu_sc as plsc

_N = 10000
_E = 320000
_D = 128
_NP = 10240
_NC = 2
_NS = 16
_NW = _NC * _NS
_EPW = _E // _NW
_K = 40
_NCHUNK = _EPW // _K
_RPT = _NP // _NS
_ZR = 16


def _sc_agg_body(want_deg, *refs):
    if want_deg:
        (h_hbm, src_hbm, dst_hbm, agg_hbm, deg_hbm,
         srcb, dstb, rows0, rows1, rows2, rows3, rows4, ones, zrows,
         sh_agg, sh_deg,
         gsem0, gsem1, gsem2, gsem3, gsem4,
         ssem0, ssem1, ssem2, ssem3, ssem4) = refs
    else:
        (h_hbm, src_hbm, dst_hbm, agg_hbm,
         srcb, dstb, rows0, rows1, rows2, rows3, rows4, ones, zrows,
         sh_agg, sh_deg,
         gsem0, gsem1, gsem2, gsem3, gsem4,
         ssem0, ssem1, ssem2, ssem3, ssem4) = refs
    c = lax.axis_index("c")
    s = lax.axis_index("s")
    wid = s * _NC + c

    zro = jnp.zeros((16,), jnp.float32)
    one = jnp.ones((16,), jnp.float32)
    for j in range((_K + 15) // 16):
        ones[pl.ds(min(j * 16, _K - 16), 16)] = one

    def _zrow(i, carry):
        for j in range(_D // 16):
            zrows[i, pl.ds(j * 16, 16)] = zro
        return carry
    lax.fori_loop(0, _ZR, _zrow, 0)

    pltpu.sync_copy(src_hbm.at[pl.ds(wid * _EPW, _EPW)], srcb)
    pltpu.sync_copy(dst_hbm.at[pl.ds(wid * _EPW, _EPW)], dstb)

    row0 = s * _RPT
    for r in range(_RPT // _ZR):
        pltpu.sync_copy(zrows, sh_agg.at[pl.ds(row0 + r * _ZR, _ZR)])
    if want_deg:
        for j in range(_RPT // _D):
            pltpu.sync_copy(zrows.at[0], sh_deg.at[pl.ds(row0 + j * _D, _D)])
    plsc.subcore_barrier()

    rows = (rows0, rows1, rows2, rows3, rows4)
    gsems = (gsem0, gsem1, gsem2, gsem3, gsem4)
    ssems = (ssem0, ssem1, ssem2, ssem3, ssem4)

    def _sidx(g):
        return srcb.at[pl.ds(g * _K, _K)]

    def _didx(g):
        return dstb.at[pl.ds(g * _K, _K)]

    def _gather(g, b):
        pltpu.async_copy(h_hbm.at[_sidx(g)], rows[b], gsems[b])

    def _wait_gather(g, b):
        pltpu.make_async_copy(h_hbm.at[_sidx(g)], rows[b], gsems[b]).wait()

    def _scatter(g, b):
        pltpu.async_copy(rows[b], sh_agg.at[_didx(g)], ssems[b], add=True)
        if want_deg:
            pltpu.sync_copy(ones, sh_deg.at[_didx(g)], add=True)

    def _wait_scatter(g, b):
        pltpu.make_async_copy(rows[b], sh_agg.at[_didx(g)], ssems[b]).wait()

    _gather(0, 0)
    _gather(1, 1)

    def _run():
        def body(j, carry):
            for b in range(5):
                g = j * 5 + b
                b2 = (b + 2) % 5
                @pl.when(g >= 3)
                def _():
                    _wait_scatter(g - 3, b2)
                @pl.when(g + 2 <= _NCHUNK - 1)
                def _():
                    _gather(g + 2, b2)
                _wait_gather(g, b)
                _scatter(g, b)
            return carry
        lax.fori_loop(0, _NCHUNK // 5, body, 0)
        for t in range(3):
            g = _NCHUNK - 3 + t
            _wait_scatter(g, g % 5)

    _run()
    plsc.subcore_barrier()

    pltpu.sync_copy(sh_agg.at[pl.ds(row0, _RPT)], agg_hbm.at[c, pl.ds(row0, _RPT)])
    if want_deg:
        pltpu.sync_copy(sh_deg.at[pl.ds(row0, _RPT)], deg_hbm.at[c, pl.ds(row0, _RPT)])


def _make_sc_agg(want_deg):
    body = functools.partial(_sc_agg_body, want_deg)
    out_type = (jax.ShapeDtypeStruct((_NC, _NP, _D), jnp.float32),
                jax.ShapeDtypeStruct((_NC, _NP), jnp.float32))
    if not want_deg:
        out_type = out_type[0]
    return functools.partial(
        pl.kernel,
        out_type=out_type,
        mesh=plsc.VectorSubcoreMesh(core_axis_name="c", subcore_axis_name="s"),
        scratch_types=[
            pltpu.VMEM((_EPW,), jnp.int32),
            pltpu.VMEM((_EPW,), jnp.int32),
            pltpu.VMEM((_K, _D), jnp.float32),
            pltpu.VMEM((_K, _D), jnp.float32),
            pltpu.VMEM((_K, _D), jnp.float32),
            pltpu.VMEM((_K, _D), jnp.float32),
            pltpu.VMEM((_K, _D), jnp.float32),
            pltpu.VMEM((_K,), jnp.float32),
            pltpu.VMEM((_ZR, _D), jnp.float32),
            pltpu.VMEM_SHARED((_NP, _D), jnp.float32),
            pltpu.VMEM_SHARED((_NP,), jnp.float32),
        ] + [pltpu.SemaphoreType.DMA] * 10,
    )(body)


_sc_agg = _make_sc_agg(True)
_sc_agg_nodeg = _make_sc_agg(False)


_BN = 1000


def _tc_pre(h, Ws, b):
    def body(h_ref, Ws_ref, b_ref, o_ref):
        o_ref[...] = (jnp.dot(h_ref[...], Ws_ref[...],
                              preferred_element_type=jnp.float32) + b_ref[...])

    return pl.pallas_call(
        body,
        grid=(_N // _BN,),
        in_specs=[
            pl.BlockSpec((_BN, _D), lambda i: (i, 0)),
            pl.BlockSpec((_D, _D), lambda i: (0, 0)),
            pl.BlockSpec((1, _D), lambda i: (0, 0)),
        ],
        out_specs=pl.BlockSpec((_BN, _D), lambda i: (i, 0)),
        out_shape=jax.ShapeDtypeStruct((_N, _D), jnp.float32),
    )(h, Ws, b)


def _tc_post(t, agg, dA, dB, Wn, relu):
    def body(t_ref, a0_ref, a1_ref, dA_ref, dB_ref, Wn_ref, o_ref):
        deg = jnp.maximum(dA_ref[...] + dB_ref[...], 1.0)
        hn = (a0_ref[0] + a1_ref[0]) / deg
        out = t_ref[...] + jnp.dot(hn, Wn_ref[...],
                                   preferred_element_type=jnp.float32)
        if relu:
            out = jnp.maximum(out, 0.0)
        o_ref[...] = out

    return pl.pallas_call(
        body,
        grid=(_N // _BN,),
        in_specs=[
            pl.BlockSpec((_BN, _D), lambda i: (i, 0)),
            pl.BlockSpec((1, _BN, _D), lambda i: (0, i, 0)),
            pl.BlockSpec((1, _BN, _D), lambda i: (1, i, 0)),
            pl.BlockSpec((_BN, 1), lambda i: (i, 0)),
            pl.BlockSpec((_BN, 1), lambda i: (i, 0)),
            pl.BlockSpec((_D, _D), lambda i: (0, 0)),
        ],
        out_specs=pl.BlockSpec((_BN, _D), lambda i: (i, 0)),
        out_shape=jax.ShapeDtypeStruct((_N, _D), jnp.float32),
    )(t, agg, agg, dA, dB, Wn)


def kernel(x, edge_index, Ws1, Wn1, b1, Ws2, Wn2, b2):
    src = edge_index[0]
    dst = edge_index[1]
    agg1, deg = _sc_agg(x, src, dst)
    dA = deg[0, :_N, None]
    dB = deg[1, :_N, None]
    t1 = _tc_pre(x, Ws1, b1.reshape(1, _D))
    h1 = _tc_post(t1, agg1, dA, dB, Wn1, relu=True)
    agg2 = _sc_agg_nodeg(h1, src, dst)
    t2 = _tc_pre(h1, Ws2, b2.reshape(1, _D))
    return _tc_post(t2, agg2, dA, dB, Wn2, relu=False)


# speedup vs baseline: 1.0008x; 1.0008x over previous
import functools

import jax
import jax.numpy as jnp
from jax import lax
from jax.experimental import pallas as pl
from jax.experimental.pallas import tpu as pltpu
from jax.experimental.pallas import tpu_sc as plsc

_N = 10000
_E = 320000
_D = 128
_NP = 10240
_NC = 2
_NS = 16
_NW = _NC * _NS
_EPW = _E // _NW
_K = 40
_NCHUNK = _EPW // _K
_RPT = _NP // _NS
_ZR = 16


def _sc_agg_body(want_deg, *refs):
    if want_deg:
        (h_hbm, src_hbm, dst_hbm, agg_hbm, deg_hbm,
         srcb, dstb, rows0, rows1, rows2, rows3, rows4, ones, zrows,
         sh_agg, sh_deg,
         gsem0, gsem1, gsem2, gsem3, gsem4,
         ssem0, ssem1, ssem2, ssem3, ssem4) = refs
    else:
        (h_hbm, src_hbm, dst_hbm, agg_hbm,
         srcb, dstb, rows0, rows1, rows2, rows3, rows4, ones, zrows,
         sh_agg, sh_deg,
         gsem0, gsem1, gsem2, gsem3, gsem4,
         ssem0, ssem1, ssem2, ssem3, ssem4) = refs
    c = lax.axis_index("c")
    s = lax.axis_index("s")
    wid = s * _NC + c

    zro = jnp.zeros((16,), jnp.float32)
    one = jnp.ones((16,), jnp.float32)
    for j in range((_K + 15) // 16):
        ones[pl.ds(min(j * 16, _K - 16), 16)] = one

    def _zrow(i, carry):
        for j in range(_D // 16):
            zrows[i, pl.ds(j * 16, 16)] = zro
        return carry
    lax.fori_loop(0, _ZR, _zrow, 0)

    pltpu.sync_copy(src_hbm.at[pl.ds(wid * _EPW, _EPW)], srcb)
    pltpu.sync_copy(dst_hbm.at[pl.ds(wid * _EPW, _EPW)], dstb)

    row0 = s * _RPT
    for r in range(_RPT // _ZR):
        pltpu.sync_copy(zrows, sh_agg.at[pl.ds(row0 + r * _ZR, _ZR)])
    if want_deg:
        for j in range(_RPT // _D):
            pltpu.sync_copy(zrows.at[0], sh_deg.at[pl.ds(row0 + j * _D, _D)])
    plsc.subcore_barrier()

    rows = (rows0, rows1, rows2, rows3, rows4)
    gsems = (gsem0, gsem1, gsem2, gsem3, gsem4)
    ssems = (ssem0, ssem1, ssem2, ssem3, ssem4)

    def _sidx(g):
        return srcb.at[pl.ds(g * _K, _K)]

    def _didx(g):
        return dstb.at[pl.ds(g * _K, _K)]

    def _gather(g, b):
        pltpu.async_copy(h_hbm.at[_sidx(g)], rows[b], gsems[b])

    def _wait_gather(g, b):
        pltpu.make_async_copy(h_hbm.at[_sidx(g)], rows[b], gsems[b]).wait()

    def _scatter(g, b):
        pltpu.async_copy(rows[b], sh_agg.at[_didx(g)], ssems[b], add=True)
        if want_deg:
            pltpu.sync_copy(ones, sh_deg.at[_didx(g)], add=True)

    def _wait_scatter(g, b):
        pltpu.make_async_copy(rows[b], sh_agg.at[_didx(g)], ssems[b]).wait()

    _gather(0, 0)
    _gather(1, 1)

    def _run():
        def body(j, carry):
            for b in range(5):
                g = j * 5 + b
                b2 = (b + 2) % 5
                @pl.when(g >= 3)
                def _():
                    _wait_scatter(g - 3, b2)
                @pl.when(g + 2 <= _NCHUNK - 1)
                def _():
                    _gather(g + 2, b2)
                _wait_gather(g, b)
                _scatter(g, b)
            return carry
        lax.fori_loop(0, _NCHUNK // 5, body, 0)
        for t in range(3):
            g = _NCHUNK - 3 + t
            _wait_scatter(g, g % 5)

    _run()
    plsc.subcore_barrier()

    pltpu.sync_copy(sh_agg.at[pl.ds(row0, _RPT)], agg_hbm.at[c, pl.ds(row0, _RPT)])
    if want_deg:
        pltpu.sync_copy(sh_deg.at[pl.ds(row0, _RPT)], deg_hbm.at[c, pl.ds(row0, _RPT)])


def _make_sc_agg(want_deg):
    body = functools.partial(_sc_agg_body, want_deg)
    out_type = (jax.ShapeDtypeStruct((_NC, _NP, _D), jnp.float32),
                jax.ShapeDtypeStruct((_NC, _NP), jnp.float32))
    if not want_deg:
        out_type = out_type[0]
    return functools.partial(
        pl.kernel,
        out_type=out_type,
        mesh=plsc.VectorSubcoreMesh(core_axis_name="c", subcore_axis_name="s"),
        scratch_types=[
            pltpu.VMEM((_EPW,), jnp.int32),
            pltpu.VMEM((_EPW,), jnp.int32),
            pltpu.VMEM((_K, _D), jnp.float32),
            pltpu.VMEM((_K, _D), jnp.float32),
            pltpu.VMEM((_K, _D), jnp.float32),
            pltpu.VMEM((_K, _D), jnp.float32),
            pltpu.VMEM((_K, _D), jnp.float32),
            pltpu.VMEM((_K,), jnp.float32),
            pltpu.VMEM((_ZR, _D), jnp.float32),
            pltpu.VMEM_SHARED((_NP, _D), jnp.float32),
            pltpu.VMEM_SHARED((_NP,), jnp.float32),
        ] + [pltpu.SemaphoreType.DMA] * 10,
    )(body)


_sc_agg = _make_sc_agg(True)
_sc_agg_nodeg = _make_sc_agg(False)


_BN = 1000


def _tc_pre(h, Ws, b):
    def body(h_ref, Ws_ref, b_ref, o_ref):
        o_ref[...] = (jnp.dot(h_ref[...], Ws_ref[...],
                              preferred_element_type=jnp.float32) + b_ref[...])

    return pl.pallas_call(
        body,
        grid=(_N // _BN,),
        in_specs=[
            pl.BlockSpec((_BN, _D), lambda i: (i, 0)),
            pl.BlockSpec((_D, _D), lambda i: (0, 0)),
            pl.BlockSpec((1, _D), lambda i: (0, 0)),
        ],
        out_specs=pl.BlockSpec((_BN, _D), lambda i: (i, 0)),
        out_shape=jax.ShapeDtypeStruct((_N, _D), jnp.float32),
    )(h, Ws, b)


def _tc_post(t, agg, dA, dB, Wn, relu, bf16_copy):
    def body(t_ref, a0_ref, a1_ref, dA_ref, dB_ref, Wn_ref, *o_refs):
        deg = jnp.maximum(dA_ref[...] + dB_ref[...], 1.0)
        hn = (a0_ref[0].astype(jnp.float32)
              + a1_ref[0].astype(jnp.float32)) / deg
        out = t_ref[...] + jnp.dot(hn, Wn_ref[...],
                                   preferred_element_type=jnp.float32)
        if relu:
            out = jnp.maximum(out, 0.0)
        o_refs[0][...] = out
        if bf16_copy:
            o_refs[1][...] = out.astype(jnp.bfloat16)

    out_shape = [jax.ShapeDtypeStruct((_N, _D), jnp.float32)]
    out_specs = [pl.BlockSpec((_BN, _D), lambda i: (i, 0))]
    if bf16_copy:
        out_shape.append(jax.ShapeDtypeStruct((_N, _D), jnp.bfloat16))
        out_specs.append(pl.BlockSpec((_BN, _D), lambda i: (i, 0)))

    return pl.pallas_call(
        body,
        grid=(_N // _BN,),
        in_specs=[
            pl.BlockSpec((_BN, _D), lambda i: (i, 0)),
            pl.BlockSpec((1, _BN, _D), lambda i: (0, i, 0)),
            pl.BlockSpec((1, _BN, _D), lambda i: (1, i, 0)),
            pl.BlockSpec((_BN, 1), lambda i: (i, 0)),
            pl.BlockSpec((_BN, 1), lambda i: (i, 0)),
            pl.BlockSpec((_D, _D), lambda i: (0, 0)),
        ],
        out_specs=out_specs,
        out_shape=out_shape,
    )(t, agg, agg, dA, dB, Wn)


def _tc_post_single(*args, **kwargs):
    return _tc_post(*args, **kwargs)[0]


def kernel(x, edge_index, Ws1, Wn1, b1, Ws2, Wn2, b2):
    src = edge_index[0]
    dst = edge_index[1]
    agg1, deg = _sc_agg(x, src, dst)
    dA = deg[0, :_N, None]
    dB = deg[1, :_N, None]
    t1 = _tc_pre(x, Ws1, b1.reshape(1, _D))
    h1 = _tc_post_single(t1, agg1, dA, dB, Wn1, relu=True, bf16_copy=False)
    agg2 = _sc_agg_nodeg(h1, src, dst)
    t2 = _tc_pre(h1, Ws2, b2.reshape(1, _D))
    return _tc_post_single(t2, agg2, dA, dB, Wn2, relu=False, bf16_copy=False)


# speedup vs baseline: 1.1209x; 1.1200x over previous
import functools

import jax
import jax.numpy as jnp
from jax import lax
from jax.experimental import pallas as pl
from jax.experimental.pallas import tpu as pltpu
from jax.experimental.pallas import tpu_sc as plsc

_N = 10000
_E = 320000
_D = 128
_NP = 10240
_NC = 2
_NS = 16
_NW = _NC * _NS
_EPW = _E // _NW
_K = 40
_NCHUNK = _EPW // _K
_RPT = _NP // _NS
_ZR = 16


def _sc_agg_body(want_deg, *refs):
    if want_deg:
        (h_hbm, src_hbm, dst_hbm, agg_hbm, deg_hbm,
         srcb, dstb, rows0, rows1, rows2, rows3, rows4, ones, zrows,
         sh_agg, sh_deg,
         gsem0, gsem1, gsem2, gsem3, gsem4,
         ssem0, ssem1, ssem2, ssem3, ssem4) = refs
    else:
        (h_hbm, src_hbm, dst_hbm, agg_hbm,
         srcb, dstb, rows0, rows1, rows2, rows3, rows4, ones, zrows,
         sh_agg, sh_deg,
         gsem0, gsem1, gsem2, gsem3, gsem4,
         ssem0, ssem1, ssem2, ssem3, ssem4) = refs
    c = lax.axis_index("c")
    s = lax.axis_index("s")
    wid = s * _NC + c

    zro = jnp.zeros((16,), jnp.float32)
    one = jnp.ones((16,), jnp.float32)
    for j in range((_K + 15) // 16):
        ones[pl.ds(min(j * 16, _K - 16), 16)] = one

    def _zrow(i, carry):
        for j in range(_D // 16):
            zrows[i, pl.ds(j * 16, 16)] = zro
        return carry
    lax.fori_loop(0, _ZR, _zrow, 0)

    row0 = s * _RPT
    pltpu.async_copy(src_hbm.at[pl.ds(wid * _EPW, _EPW)], srcb, gsem0)
    pltpu.async_copy(dst_hbm.at[pl.ds(wid * _EPW, _EPW)], dstb, gsem0)
    for r in range(_RPT // _ZR):
        pltpu.async_copy(zrows, sh_agg.at[pl.ds(row0 + r * _ZR, _ZR)], gsem1)
    if want_deg:
        for j in range(_RPT // _D):
            pltpu.async_copy(zrows.at[0], sh_deg.at[pl.ds(row0 + j * _D, _D)],
                             gsem1)
    pltpu.make_async_copy(src_hbm.at[pl.ds(wid * _EPW, _EPW)], srcb, gsem0).wait()
    pltpu.make_async_copy(dst_hbm.at[pl.ds(wid * _EPW, _EPW)], dstb, gsem0).wait()
    for r in range(_RPT // _ZR):
        pltpu.make_async_copy(zrows, sh_agg.at[pl.ds(row0 + r * _ZR, _ZR)],
                              gsem1).wait()
    if want_deg:
        for j in range(_RPT // _D):
            pltpu.make_async_copy(zrows.at[0],
                                  sh_deg.at[pl.ds(row0 + j * _D, _D)],
                                  gsem1).wait()
    plsc.subcore_barrier()

    rows = (rows0, rows1, rows2, rows3, rows4)
    gsems = (gsem0, gsem1, gsem2, gsem3, gsem4)
    ssems = (ssem0, ssem1, ssem2, ssem3, ssem4)

    def _sidx(g):
        return srcb.at[pl.ds(g * _K, _K)]

    def _didx(g):
        return dstb.at[pl.ds(g * _K, _K)]

    def _gather(g, b):
        pltpu.async_copy(h_hbm.at[_sidx(g)], rows[b], gsems[b])

    def _wait_gather(g, b):
        pltpu.make_async_copy(h_hbm.at[_sidx(g)], rows[b], gsems[b]).wait()

    def _scatter(g, b):
        pltpu.async_copy(rows[b], sh_agg.at[_didx(g)], ssems[b], add=True)
        if want_deg:
            pltpu.sync_copy(ones, sh_deg.at[_didx(g)], add=True)

    def _wait_scatter(g, b):
        pltpu.make_async_copy(rows[b], sh_agg.at[_didx(g)], ssems[b]).wait()

    _gather(0, 0)
    _gather(1, 1)
    _gather(2, 2)

    def _run():
        def body(j, carry):
            for b in range(5):
                g = j * 5 + b
                b3 = (b + 3) % 5
                @pl.when(g >= 2)
                def _():
                    _wait_scatter(g - 2, b3)
                @pl.when(g + 3 <= _NCHUNK - 1)
                def _():
                    _gather(g + 3, b3)
                _wait_gather(g, b)
                _scatter(g, b)
            return carry
        lax.fori_loop(0, _NCHUNK // 5, body, 0)
        for t in range(2):
            g = _NCHUNK - 2 + t
            _wait_scatter(g, g % 5)

    _run()
    plsc.subcore_barrier()

    pltpu.sync_copy(sh_agg.at[pl.ds(row0, _RPT)], agg_hbm.at[c, pl.ds(row0, _RPT)])
    if want_deg:
        pltpu.sync_copy(sh_deg.at[pl.ds(row0, _RPT)], deg_hbm.at[c, pl.ds(row0, _RPT)])


def _make_sc_agg(want_deg):
    body = functools.partial(_sc_agg_body, want_deg)
    out_type = (jax.ShapeDtypeStruct((_NC, _NP, _D), jnp.float32),
                jax.ShapeDtypeStruct((_NC, _NP), jnp.float32))
    if not want_deg:
        out_type = out_type[0]
    return functools.partial(
        pl.kernel,
        out_type=out_type,
        mesh=plsc.VectorSubcoreMesh(core_axis_name="c", subcore_axis_name="s"),
        scratch_types=[
            pltpu.VMEM((_EPW,), jnp.int32),
            pltpu.VMEM((_EPW,), jnp.int32),
            pltpu.VMEM((_K, _D), jnp.float32),
            pltpu.VMEM((_K, _D), jnp.float32),
            pltpu.VMEM((_K, _D), jnp.float32),
            pltpu.VMEM((_K, _D), jnp.float32),
            pltpu.VMEM((_K, _D), jnp.float32),
            pltpu.VMEM((_K,), jnp.float32),
            pltpu.VMEM((_ZR, _D), jnp.float32),
            pltpu.VMEM_SHARED((_NP, _D), jnp.float32),
            pltpu.VMEM_SHARED((_NP,), jnp.float32),
        ] + [pltpu.SemaphoreType.DMA] * 10,
    )(body)


_sc_agg = _make_sc_agg(True)
_sc_agg_nodeg = _make_sc_agg(False)


_BN = 1000


def _tc_pre(h, Ws, b):
    def body(h_ref, Ws_ref, b_ref, o_ref):
        o_ref[...] = (jnp.dot(h_ref[...], Ws_ref[...],
                              preferred_element_type=jnp.float32) + b_ref[...])

    return pl.pallas_call(
        body,
        grid=(_N // _BN,),
        in_specs=[
            pl.BlockSpec((_BN, _D), lambda i: (i, 0)),
            pl.BlockSpec((_D, _D), lambda i: (0, 0)),
            pl.BlockSpec((1, _D), lambda i: (0, 0)),
        ],
        out_specs=pl.BlockSpec((_BN, _D), lambda i: (i, 0)),
        out_shape=jax.ShapeDtypeStruct((_N, _D), jnp.float32),
    )(h, Ws, b)


def _tc_post(t, agg, dA, dB, Wn, relu, bf16_copy):
    def body(t_ref, a0_ref, a1_ref, dA_ref, dB_ref, Wn_ref, *o_refs):
        deg = jnp.maximum(dA_ref[...] + dB_ref[...], 1.0)
        hn = (a0_ref[0].astype(jnp.float32)
              + a1_ref[0].astype(jnp.float32)) / deg
        out = t_ref[...] + jnp.dot(hn, Wn_ref[...],
                                   preferred_element_type=jnp.float32)
        if relu:
            out = jnp.maximum(out, 0.0)
        o_refs[0][...] = out
        if bf16_copy:
            o_refs[1][...] = out.astype(jnp.bfloat16)

    out_shape = [jax.ShapeDtypeStruct((_N, _D), jnp.float32)]
    out_specs = [pl.BlockSpec((_BN, _D), lambda i: (i, 0))]
    if bf16_copy:
        out_shape.append(jax.ShapeDtypeStruct((_N, _D), jnp.bfloat16))
        out_specs.append(pl.BlockSpec((_BN, _D), lambda i: (i, 0)))

    return pl.pallas_call(
        body,
        grid=(_N // _BN,),
        in_specs=[
            pl.BlockSpec((_BN, _D), lambda i: (i, 0)),
            pl.BlockSpec((1, _BN, _D), lambda i: (0, i, 0)),
            pl.BlockSpec((1, _BN, _D), lambda i: (1, i, 0)),
            pl.BlockSpec((_BN, 1), lambda i: (i, 0)),
            pl.BlockSpec((_BN, 1), lambda i: (i, 0)),
            pl.BlockSpec((_D, _D), lambda i: (0, 0)),
        ],
        out_specs=out_specs,
        out_shape=out_shape,
    )(t, agg, agg, dA, dB, Wn)


def _tc_post_single(*args, **kwargs):
    return _tc_post(*args, **kwargs)[0]


def kernel(x, edge_index, Ws1, Wn1, b1, Ws2, Wn2, b2):
    src = edge_index[0]
    dst = edge_index[1]
    agg1, deg = _sc_agg(x, src, dst)
    dA = deg[0, :_N, None]
    dB = deg[1, :_N, None]
    t1 = _tc_pre(x, Ws1, b1.reshape(1, _D))
    h1 = _tc_post_single(t1, agg1, dA, dB, Wn1, relu=True, bf16_copy=False)
    agg2 = _sc_agg_nodeg(h1, src, dst)
    t2 = _tc_pre(h1, Ws2, b2.reshape(1, _D))
    return _tc_post_single(t2, agg2, dA, dB, Wn2, relu=False, bf16_copy=False)


# speedup vs baseline: 1.1327x; 1.0105x over previous
import functools

import jax
import jax.numpy as jnp
from jax import lax
from jax.experimental import pallas as pl
from jax.experimental.pallas import tpu as pltpu
from jax.experimental.pallas import tpu_sc as plsc

_N = 10000
_E = 320000
_D = 128
_NP = 10240
_NC = 2
_NS = 16
_NW = _NC * _NS
_EPW = _E // _NW
_K = 40
_NCHUNK = _EPW // _K
_RPT = _NP // _NS
_ZR = 16


def _sc_agg_body(want_deg, *refs):
    if want_deg:
        (h_hbm, src_hbm, dst_hbm, agg_hbm, deg_hbm,
         srcb, dstb, rows0, rows1, rows2, rows3, rows4, ones, zrows,
         sh_agg, sh_deg,
         gsem0, gsem1, gsem2, gsem3, gsem4,
         ssem0, ssem1, ssem2, ssem3, ssem4, osem) = refs
    else:
        (h_hbm, src_hbm, dst_hbm, agg_hbm,
         srcb, dstb, rows0, rows1, rows2, rows3, rows4, ones, zrows,
         sh_agg, sh_deg,
         gsem0, gsem1, gsem2, gsem3, gsem4,
         ssem0, ssem1, ssem2, ssem3, ssem4, osem) = refs
    c = lax.axis_index("c")
    s = lax.axis_index("s")
    wid = s * _NC + c

    zro = jnp.zeros((16,), jnp.float32)
    one = jnp.ones((16,), jnp.float32)
    for j in range((_K + 15) // 16):
        ones[pl.ds(min(j * 16, _K - 16), 16)] = one

    def _zrow(i, carry):
        for j in range(_D // 16):
            zrows[i, pl.ds(j * 16, 16)] = zro
        return carry
    lax.fori_loop(0, _ZR, _zrow, 0)

    row0 = s * _RPT
    pltpu.async_copy(src_hbm.at[pl.ds(wid * _EPW, _EPW)], srcb, gsem0)
    pltpu.async_copy(dst_hbm.at[pl.ds(wid * _EPW, _EPW)], dstb, gsem0)
    for r in range(_RPT // _ZR):
        pltpu.async_copy(zrows, sh_agg.at[pl.ds(row0 + r * _ZR, _ZR)], ssem0)
    if want_deg:
        for j in range(_RPT // _D):
            pltpu.async_copy(zrows.at[0], sh_deg.at[pl.ds(row0 + j * _D, _D)],
                             ssem1)
    pltpu.make_async_copy(src_hbm.at[pl.ds(wid * _EPW, _EPW)], srcb, gsem0).wait()
    pltpu.make_async_copy(dst_hbm.at[pl.ds(wid * _EPW, _EPW)], dstb, gsem0).wait()
    pltpu.async_copy(h_hbm.at[srcb.at[pl.ds(0 * _K, _K)]], rows0, gsem0)
    pltpu.async_copy(h_hbm.at[srcb.at[pl.ds(1 * _K, _K)]], rows1, gsem1)
    pltpu.async_copy(h_hbm.at[srcb.at[pl.ds(2 * _K, _K)]], rows2, gsem2)
    for r in range(_RPT // _ZR):
        pltpu.make_async_copy(zrows, sh_agg.at[pl.ds(row0 + r * _ZR, _ZR)],
                              ssem0).wait()
    if want_deg:
        for j in range(_RPT // _D):
            pltpu.make_async_copy(zrows.at[0],
                                  sh_deg.at[pl.ds(row0 + j * _D, _D)],
                                  ssem1).wait()
    plsc.subcore_barrier()

    rows = (rows0, rows1, rows2, rows3, rows4)
    gsems = (gsem0, gsem1, gsem2, gsem3, gsem4)
    ssems = (ssem0, ssem1, ssem2, ssem3, ssem4)

    def _sidx(g):
        return srcb.at[pl.ds(g * _K, _K)]

    def _didx(g):
        return dstb.at[pl.ds(g * _K, _K)]

    def _gather(g, b):
        pltpu.async_copy(h_hbm.at[_sidx(g)], rows[b], gsems[b])

    def _wait_gather(g, b):
        pltpu.make_async_copy(h_hbm.at[_sidx(g)], rows[b], gsems[b]).wait()

    def _scatter(g, b):
        pltpu.async_copy(rows[b], sh_agg.at[_didx(g)], ssems[b], add=True)
        if want_deg:
            pltpu.async_copy(ones, sh_deg.at[_didx(g)], osem, add=True)

    def _wait_deg(g):
        if want_deg:
            pltpu.make_async_copy(ones, sh_deg.at[_didx(g)], osem).wait()

    def _wait_scatter(g, b):
        pltpu.make_async_copy(rows[b], sh_agg.at[_didx(g)], ssems[b]).wait()

    def _run():
        def body(j, carry):
            for b in range(5):
                g = j * 5 + b
                b3 = (b + 3) % 5
                @pl.when(g >= 2)
                def _():
                    _wait_scatter(g - 2, b3)
                    _wait_deg(g - 2)
                @pl.when(g + 3 <= _NCHUNK - 1)
                def _():
                    _gather(g + 3, b3)
                _wait_gather(g, b)
                _scatter(g, b)
            return carry
        lax.fori_loop(0, _NCHUNK // 5, body, 0)
        for t in range(2):
            g = _NCHUNK - 2 + t
            _wait_scatter(g, g % 5)
            _wait_deg(g)

    _run()
    plsc.subcore_barrier()

    pltpu.async_copy(sh_agg.at[pl.ds(row0, _RPT)], agg_hbm.at[c, pl.ds(row0, _RPT)],
                     gsem0)
    if want_deg:
        pltpu.async_copy(sh_deg.at[pl.ds(row0, _RPT)],
                         deg_hbm.at[c, pl.ds(row0, _RPT)], gsem1)
    pltpu.make_async_copy(sh_agg.at[pl.ds(row0, _RPT)],
                          agg_hbm.at[c, pl.ds(row0, _RPT)], gsem0).wait()
    if want_deg:
        pltpu.make_async_copy(sh_deg.at[pl.ds(row0, _RPT)],
                              deg_hbm.at[c, pl.ds(row0, _RPT)], gsem1).wait()


def _make_sc_agg(want_deg):
    body = functools.partial(_sc_agg_body, want_deg)
    out_type = (jax.ShapeDtypeStruct((_NC, _NP, _D), jnp.float32),
                jax.ShapeDtypeStruct((_NC, _NP), jnp.float32))
    if not want_deg:
        out_type = out_type[0]
    return functools.partial(
        pl.kernel,
        out_type=out_type,
        mesh=plsc.VectorSubcoreMesh(core_axis_name="c", subcore_axis_name="s"),
        scratch_types=[
            pltpu.VMEM((_EPW,), jnp.int32),
            pltpu.VMEM((_EPW,), jnp.int32),
            pltpu.VMEM((_K, _D), jnp.float32),
            pltpu.VMEM((_K, _D), jnp.float32),
            pltpu.VMEM((_K, _D), jnp.float32),
            pltpu.VMEM((_K, _D), jnp.float32),
            pltpu.VMEM((_K, _D), jnp.float32),
            pltpu.VMEM((_K,), jnp.float32),
            pltpu.VMEM((_ZR, _D), jnp.float32),
            pltpu.VMEM_SHARED((_NP, _D), jnp.float32),
            pltpu.VMEM_SHARED((_NP,), jnp.float32),
        ] + [pltpu.SemaphoreType.DMA] * 11,
    )(body)


_sc_agg = _make_sc_agg(True)
_sc_agg_nodeg = _make_sc_agg(False)


_BN = 1000


def _tc_pre(h, Ws, b):
    def body(h_ref, Ws_ref, b_ref, o_ref):
        o_ref[...] = (jnp.dot(h_ref[...], Ws_ref[...],
                              preferred_element_type=jnp.float32) + b_ref[...])

    return pl.pallas_call(
        body,
        grid=(_N // _BN,),
        in_specs=[
            pl.BlockSpec((_BN, _D), lambda i: (i, 0)),
            pl.BlockSpec((_D, _D), lambda i: (0, 0)),
            pl.BlockSpec((1, _D), lambda i: (0, 0)),
        ],
        out_specs=pl.BlockSpec((_BN, _D), lambda i: (i, 0)),
        out_shape=jax.ShapeDtypeStruct((_N, _D), jnp.float32),
    )(h, Ws, b)


def _tc_post(t, agg, dA, dB, Wn, relu, bf16_copy):
    def body(t_ref, a0_ref, a1_ref, dA_ref, dB_ref, Wn_ref, *o_refs):
        deg = jnp.maximum(dA_ref[...] + dB_ref[...], 1.0)
        hn = (a0_ref[0].astype(jnp.float32)
              + a1_ref[0].astype(jnp.float32)) / deg
        out = t_ref[...] + jnp.dot(hn, Wn_ref[...],
                                   preferred_element_type=jnp.float32)
        if relu:
            out = jnp.maximum(out, 0.0)
        o_refs[0][...] = out
        if bf16_copy:
            o_refs[1][...] = out.astype(jnp.bfloat16)

    out_shape = [jax.ShapeDtypeStruct((_N, _D), jnp.float32)]
    out_specs = [pl.BlockSpec((_BN, _D), lambda i: (i, 0))]
    if bf16_copy:
        out_shape.append(jax.ShapeDtypeStruct((_N, _D), jnp.bfloat16))
        out_specs.append(pl.BlockSpec((_BN, _D), lambda i: (i, 0)))

    return pl.pallas_call(
        body,
        grid=(_N // _BN,),
        in_specs=[
            pl.BlockSpec((_BN, _D), lambda i: (i, 0)),
            pl.BlockSpec((1, _BN, _D), lambda i: (0, i, 0)),
            pl.BlockSpec((1, _BN, _D), lambda i: (1, i, 0)),
            pl.BlockSpec((_BN, 1), lambda i: (i, 0)),
            pl.BlockSpec((_BN, 1), lambda i: (i, 0)),
            pl.BlockSpec((_D, _D), lambda i: (0, 0)),
        ],
        out_specs=out_specs,
        out_shape=out_shape,
    )(t, agg, agg, dA, dB, Wn)


def _tc_post_single(*args, **kwargs):
    return _tc_post(*args, **kwargs)[0]


def kernel(x, edge_index, Ws1, Wn1, b1, Ws2, Wn2, b2):
    src = edge_index[0]
    dst = edge_index[1]
    agg1, deg = _sc_agg(x, src, dst)
    dA = deg[0, :_N, None]
    dB = deg[1, :_N, None]
    t1 = _tc_pre(x, Ws1, b1.reshape(1, _D))
    h1 = _tc_post_single(t1, agg1, dA, dB, Wn1, relu=True, bf16_copy=False)
    agg2 = _sc_agg_nodeg(h1, src, dst)
    t2 = _tc_pre(h1, Ws2, b2.reshape(1, _D))
    return _tc_post_single(t2, agg2, dA, dB, Wn2, relu=False, bf16_copy=False)
